# Initial kernel scaffold; baseline (speedup 1.0000x reference)
import jax
import jax.numpy as jnp
from jax.experimental import pallas as pl


def kernel(x, edge_index, W1, b1, W2, b2, W3, b3, Wm, bm, gamma, beta):
    raise NotImplementedError("write your pallas kernel here")


# speedup vs baseline: 9.1357x; 9.1357x over previous
import functools

import jax
import jax.numpy as jnp
from jax import lax
from jax.experimental import pallas as pl
from jax.experimental.pallas import tpu as pltpu
from jax.experimental.pallas import tpu_sc as plsc

NSUB = 16
NCORE = 2
NW = NSUB * NCORE
CHUNK = 128
LANES = 16
ZR = 64


def _make_deg_kernel(nch, npd):
    rpt = npd // NSUB
    mesh = plsc.VectorSubcoreMesh(core_axis_name="c", subcore_axis_name="s")

    @functools.partial(
        pl.kernel,
        out_type=jax.ShapeDtypeStruct((NCORE, npd), jnp.float32),
        mesh=mesh,
        scratch_types=[
            pltpu.VMEM((nch, CHUNK), jnp.int32),
            pltpu.VMEM((CHUNK,), jnp.float32),
            pltpu.VMEM((rpt,), jnp.float32),
            pltpu.VMEM_SHARED((npd,), jnp.float32),
        ],
    )
    def deg_kernel(dstp_hbm, out_hbm, dst_v, ones_v, zb_v, acc_sh):
        cid = lax.axis_index("c")
        sid = lax.axis_index("s")
        wid = cid * NSUB + sid
        pltpu.sync_copy(dstp_hbm.at[wid], dst_v)
        onev = jnp.ones((LANES,), jnp.float32)
        zerov = jnp.zeros((LANES,), jnp.float32)
        for k in range(CHUNK // LANES):
            ones_v[pl.ds(k * LANES, LANES)] = onev

        def zrow(r, carry):
            zb_v[pl.ds(r * LANES, LANES)] = zerov
            return carry

        lax.fori_loop(0, rpt // LANES, zrow, 0)
        pltpu.sync_copy(zb_v, acc_sh.at[pl.ds(sid * rpt, rpt)])
        plsc.subcore_barrier()

        def step(j, carry):
            pltpu.sync_copy(ones_v, acc_sh.at[dst_v.at[j]], add=True)
            return carry

        lax.fori_loop(0, nch, step, 0)
        plsc.subcore_barrier()
        pltpu.sync_copy(acc_sh.at[pl.ds(sid * rpt, rpt)],
                        out_hbm.at[cid, pl.ds(sid * rpt, rpt)])

    return deg_kernel


def _make_edge_kernel(nch, npd, h):
    rpt = npd // NSUB
    mesh = plsc.VectorSubcoreMesh(core_axis_name="c", subcore_axis_name="s")

    @functools.partial(
        pl.kernel,
        out_type=jax.ShapeDtypeStruct((NCORE, npd, h), jnp.float32),
        mesh=mesh,
        scratch_types=[
            pltpu.VMEM((nch, CHUNK), jnp.int32),
            pltpu.VMEM((nch, CHUNK), jnp.int32),
            pltpu.VMEM((CHUNK, h), jnp.float32),
            pltpu.VMEM((ZR, h), jnp.float32),
            pltpu.VMEM_SHARED((npd, h), jnp.float32),
            pltpu.SemaphoreType.DMA,
        ],
    )
    def edge_kernel(hs_hbm, srcp_hbm, dstp_hbm, out_hbm,
                    src_v, dst_v, rows_v, zb_v, acc_sh, sem):
        cid = lax.axis_index("c")
        sid = lax.axis_index("s")
        wid = cid * NSUB + sid
        pltpu.sync_copy(srcp_hbm.at[wid], src_v)
        pltpu.sync_copy(dstp_hbm.at[wid], dst_v)

        zerov = jnp.zeros((LANES,), jnp.float32)

        def zrow(r, carry):
            for c in range(h // LANES):
                zb_v[r, pl.ds(c * LANES, LANES)] = zerov
            return carry

        lax.fori_loop(0, ZR, zrow, 0)
        for k in range(rpt // ZR):
            pltpu.sync_copy(zb_v, acc_sh.at[pl.ds(sid * rpt + k * ZR, ZR)])
        plsc.subcore_barrier()

        def step(j, carry):
            pltpu.async_copy(hs_hbm.at[src_v.at[j]], rows_v, sem).wait()
            pltpu.sync_copy(rows_v, acc_sh.at[dst_v.at[j]], add=True)
            return carry

        lax.fori_loop(0, nch, step, 0)
        plsc.subcore_barrier()
        pltpu.sync_copy(acc_sh.at[pl.ds(sid * rpt, rpt)],
                        out_hbm.at[cid, pl.ds(sid * rpt, rpt)])

    return edge_kernel


def _gelu(v):
    return 0.5 * v * (1.0 + lax.erf(v * 0.7071067811865476))


def _k1_body(degp_ref, x_ref, w_ref, hs_ref, dinv_ref):
    deg = degp_ref[0] + degp_ref[1] + 1.0
    dinv = lax.rsqrt(deg)
    hs_ref[...] = jnp.dot(x_ref[...], w_ref[...],
                          preferred_element_type=jnp.float32) * dinv
    dinv_ref[...] = dinv


def _mid_body(p_ref, hs_ref, dinv_ref, b_ref, w_ref, g_ref, hsn_ref):
    dinv = dinv_ref[...]
    g = _gelu(dinv * (p_ref[0] + p_ref[1] + hs_ref[...]) + b_ref[...])
    g_ref[...] = g
    hsn_ref[...] = jnp.dot(g, w_ref[...],
                           preferred_element_type=jnp.float32) * dinv


def _final_body(x_ref, g1_ref, g2_ref, g3_ref, wm_ref, bm_ref, gamma_ref,
                beta_ref, out_ref):
    z = jnp.dot(x_ref[...], wm_ref[0], preferred_element_type=jnp.float32)
    z += jnp.dot(g1_ref[...], wm_ref[1], preferred_element_type=jnp.float32)
    z += jnp.dot(g2_ref[...], wm_ref[2], preferred_element_type=jnp.float32)
    z += jnp.dot(g3_ref[...], wm_ref[3], preferred_element_type=jnp.float32)
    o = _gelu(z + bm_ref[...])
    mu = jnp.mean(o, axis=-1, keepdims=True)
    var = jnp.mean((o - mu) ** 2, axis=-1, keepdims=True)
    out_ref[...] = (o - mu) * lax.rsqrt(var + 1e-5) * gamma_ref[...] + beta_ref[...]


def kernel(x, edge_index, W1, b1, W2, b2, W3, b3, Wm, bm, gamma, beta):
    n, d = x.shape
    h = W1.shape[1]
    e = edge_index.shape[1]

    nch = -(-e // (NW * CHUNK))
    ep = NW * nch * CHUNK
    npd = -(-(n + 1) // (NSUB * ZR)) * (NSUB * ZR)

    src = edge_index[0]
    dst = edge_index[1]
    pad = ep - e
    srcp = jnp.concatenate([src, jnp.zeros((pad,), jnp.int32)]).reshape(NW, nch, CHUNK)
    dstp = jnp.concatenate([dst, jnp.full((pad,), n, jnp.int32)]).reshape(NW, nch, CHUNK)

    deg_kernel = _make_deg_kernel(nch, npd)
    edge_kernel = _make_edge_kernel(nch, npd, h)

    degp = deg_kernel(dstp)
    degp3 = degp.reshape(NCORE, npd, 1)

    bn = 2000
    grid = (n // bn,)

    def rows2(i):
        return (i, 0)

    full2 = pl.BlockSpec((d, h), lambda i: (0, 0))
    row_spec = pl.BlockSpec((bn, h), rows2)
    col_spec = pl.BlockSpec((bn, 1), rows2)
    part_spec = pl.BlockSpec((NCORE, bn, h), lambda i: (0, i, 0))
    vec_spec = pl.BlockSpec((1, h), lambda i: (0, 0))

    hs1, dinv = pl.pallas_call(
        _k1_body,
        grid=grid,
        in_specs=[pl.BlockSpec((NCORE, bn, 1), lambda i: (0, i, 0)),
                  row_spec, full2],
        out_specs=[row_spec, col_spec],
        out_shape=[jax.ShapeDtypeStruct((n, h), jnp.float32),
                   jax.ShapeDtypeStruct((n, 1), jnp.float32)],
    )(degp3, x, W1)

    mid_call = pl.pallas_call(
        _mid_body,
        grid=grid,
        in_specs=[part_spec, row_spec, col_spec, vec_spec, full2],
        out_specs=[row_spec, row_spec],
        out_shape=[jax.ShapeDtypeStruct((n, h), jnp.float32),
                   jax.ShapeDtypeStruct((n, h), jnp.float32)],
    )

    def hop(hs, wb):
        w_next, b = wb
        p = edge_kernel(hs, srcp, dstp)
        g, hs_next = mid_call(p, hs, dinv, b, w_next)
        return hs_next, g

    ws = jnp.stack([W2, W3, W3])
    bs = jnp.stack([b1.reshape(1, h), b2.reshape(1, h), b3.reshape(1, h)])
    _, gs = lax.scan(hop, hs1, (ws, bs))

    out = pl.pallas_call(
        _final_body,
        grid=grid,
        in_specs=[row_spec, row_spec, row_spec, row_spec,
                  pl.BlockSpec((4, d, h), lambda i: (0, 0, 0)),
                  vec_spec, vec_spec, vec_spec],
        out_specs=row_spec,
        out_shape=jax.ShapeDtypeStruct((n, h), jnp.float32),
    )(x, gs[0], gs[1], gs[2],
      Wm.reshape(4, d, h), bm.reshape(1, h), gamma.reshape(1, h),
      beta.reshape(1, h))
    return out
